# baseline (speedup 1.0000x reference)
import functools

import jax
import jax.numpy as jnp
from jax import lax
from jax.experimental import pallas as pl
from jax.experimental.pallas import tpu as pltpu
from jax.experimental.pallas import tpu_sc as plsc

_DPAD = 384
_NT = _DPAD // 128
_CHUNK = 40


@functools.lru_cache(maxsize=None)
def _make_gather(n_total: int):
    info = plsc.get_sparse_core_info()
    nc = info.num_cores
    nw = nc * info.num_subcores
    per_w = n_total // nw
    n_chunks = per_w // _CHUNK

    mesh = plsc.VectorSubcoreMesh(core_axis_name="c", subcore_axis_name="s")

    @functools.partial(
        pl.kernel,
        mesh=mesh,
        compiler_params=pltpu.CompilerParams(needs_layout_passes=False),
        out_type=jax.ShapeDtypeStruct((_NT * n_total, 128), jnp.float32),
        scratch_types=[
            pltpu.VMEM((per_w,), jnp.int32),
            pltpu.VMEM((2, _NT * _CHUNK), jnp.int32),
            pltpu.VMEM((2, _NT * _CHUNK, 128), jnp.float32),
            pltpu.SemaphoreType.DMA,
            pltpu.SemaphoreType.DMA,
            pltpu.SemaphoreType.DMA,
            pltpu.SemaphoreType.DMA,
        ],
    )
    def gather_kernel(table_hbm, idx_hbm, out_hbm, idx_v, jb, rows,
                      sg0, sg1, so0, so1):
        wid = lax.axis_index("s") * nc + lax.axis_index("c")
        base = wid * per_w
        sg = (sg0, sg1)
        so = (so0, so1)

        pltpu.sync_copy(idx_hbm.at[pl.ds(base, per_w)], idx_v)

        iota = lax.iota(jnp.int32, 16)
        i3 = iota * _NT

        def build(c, p):
            for off in (0, 16, _CHUNK - 16):
                ids = idx_v[pl.ds(c * _CHUNK + off, 16)]
                v3 = ids * _NT
                for t in range(_NT):
                    plsc.store_scatter(jb.at[p], [i3 + (_NT * off + t)],
                                       v3 + t)

        def fire(p):
            return pltpu.async_copy(table_hbm.at[jb.at[p]], rows.at[p], sg[p])

        gathers = [None, None]
        outs = [None, None]
        build(0, 0)
        gathers[0] = fire(0)
        for c in range(n_chunks):
            p = c % 2
            q = (c + 1) % 2
            if c + 1 < n_chunks:
                build(c + 1, q)
            gathers[p].wait()
            if c + 1 < n_chunks:
                if outs[q] is not None:
                    outs[q].wait()
                gathers[q] = fire(q)
            outs[p] = pltpu.async_copy(
                rows.at[p],
                out_hbm.at[pl.ds(_NT * (base + c * _CHUNK), _NT * _CHUNK)],
                so[p])
        for o in outs:
            if o is not None:
                o.wait()

    return gather_kernel


def kernel(table, word_ids):
    b, l = word_ids.shape
    vocab, dim = table.shape
    n = b * l
    idx = word_ids.reshape(-1).astype(jnp.int32)
    t3 = jnp.pad(table, ((0, 0), (0, _DPAD - dim))).reshape(_NT * vocab, 128)
    h = n // 2
    g = _make_gather(h)
    s0 = g(t3, idx[:h])
    s1 = g(t3, idx[h:])
    a0 = s0.reshape(b // 2, l, _DPAD)[:, :, :dim]
    a1 = s1.reshape(b // 2, l, _DPAD)[:, :, :dim]
    return jnp.concatenate([a0, a1], axis=0)


# speedup vs baseline: 1.1917x; 1.1917x over previous
import functools

import jax
import jax.numpy as jnp
from jax import lax
from jax.experimental import pallas as pl
from jax.experimental.pallas import tpu as pltpu
from jax.experimental.pallas import tpu_sc as plsc

_DPAD = 384
_NT = _DPAD // 128
_CHUNK = 40


@functools.lru_cache(maxsize=None)
def _make_gather(n_total: int):
    info = plsc.get_sparse_core_info()
    nc = info.num_cores
    nw = nc * info.num_subcores
    per_w = n_total // nw
    n_chunks = per_w // _CHUNK

    mesh = plsc.VectorSubcoreMesh(core_axis_name="c", subcore_axis_name="s")

    @functools.partial(
        pl.kernel,
        mesh=mesh,
        compiler_params=pltpu.CompilerParams(needs_layout_passes=False),
        out_type=jax.ShapeDtypeStruct((_NT * n_total, 128), jnp.float32),
        scratch_types=[
            pltpu.VMEM((per_w,), jnp.int32),
            pltpu.VMEM((2, _NT * _CHUNK), jnp.int32),
            pltpu.VMEM((2, _NT * _CHUNK, 128), jnp.float32),
            pltpu.SemaphoreType.DMA,
            pltpu.SemaphoreType.DMA,
            pltpu.SemaphoreType.DMA,
            pltpu.SemaphoreType.DMA,
        ],
    )
    def gather_kernel(table_hbm, idx_hbm, out_hbm, idx_v, jb, rows,
                      sg0, sg1, so0, so1):
        wid = lax.axis_index("s") * nc + lax.axis_index("c")
        base = wid * per_w
        sg = (sg0, sg1)
        so = (so0, so1)

        pltpu.sync_copy(idx_hbm.at[pl.ds(base, per_w)], idx_v)

        iota = lax.iota(jnp.int32, 16)
        i3 = iota * _NT

        def build(c, p):
            for off in (0, 16, _CHUNK - 16):
                ids = idx_v[pl.ds(c * _CHUNK + off, 16)]
                v3 = ids * _NT
                for t in range(_NT):
                    plsc.store_scatter(jb.at[p], [i3 + (_NT * off + t)],
                                       v3 + t)

        def fire(p):
            return pltpu.async_copy(table_hbm.at[jb.at[p]], rows.at[p], sg[p])

        gathers = [None, None]
        outs = [None, None]
        build(0, 0)
        gathers[0] = fire(0)
        for c in range(n_chunks):
            p = c % 2
            q = (c + 1) % 2
            if c + 1 < n_chunks:
                build(c + 1, q)
            gathers[p].wait()
            if c + 1 < n_chunks:
                if outs[q] is not None:
                    outs[q].wait()
                gathers[q] = fire(q)
            outs[p] = pltpu.async_copy(
                rows.at[p],
                out_hbm.at[pl.ds(_NT * (base + c * _CHUNK), _NT * _CHUNK)],
                so[p])
        for o in outs:
            if o is not None:
                o.wait()

    return gather_kernel


def kernel(table, word_ids):
    b, l = word_ids.shape
    vocab, dim = table.shape
    n = b * l
    idx = word_ids.reshape(-1).astype(jnp.int32)
    t3 = jnp.pad(table, ((0, 0), (0, _DPAD - dim))).reshape(_NT * vocab, 128)
    staged = _make_gather(n)(t3, idx)
    return staged.reshape(b, l, _DPAD)[:, :, :dim]
